# baseline (speedup 1.0000x reference)
import functools

import numpy as np
import jax
import jax.numpy as jnp
from jax import lax
from jax.experimental import pallas as pl
from jax.experimental.pallas import tpu as pltpu

_BINS = 10
_MOM = np.float32(0.75)


def _ghm_body(logits_ref, labels_ref, acc_ref, out_ref, cacc, sacc,
              *, nblk, rows, ncls, total):
    i = pl.program_id(0)

    @pl.when(i == 0)
    def _init():
        for k in range(_BINS):
            cacc[k] = np.float32(0)
            sacc[k] = np.float32(0)

    x = logits_ref[...]
    lab = labels_ref[0, 0, :]
    m = jnp.max(x, axis=1)
    e = jnp.exp(x - m[:, None])
    z = jnp.sum(e, axis=1)
    col = lax.broadcasted_iota(jnp.int32, (rows, ncls), 1)
    sel = col == lab[:, None]
    xl = jnp.sum(jnp.where(sel, x, np.float32(0)), axis=1)
    u = xl - m
    ce = jnp.log(z) - u
    g = np.float32(1) - jnp.exp(u) / z
    binv = jnp.zeros(g.shape, jnp.int32)
    for j in range(_BINS):
        binv = binv + (g > np.float32(j) / np.float32(10)).astype(jnp.int32)
    for k in range(_BINS):
        mk = binv == k
        cacc[k] = cacc[k] + jnp.sum(mk.astype(jnp.float32))
        sacc[k] = sacc[k] + jnp.sum(jnp.where(mk, ce, np.float32(0)))

    @pl.when(i == nblk - 1)
    def _finish():
        loss = np.float32(0)
        wsum = np.float32(0)
        for k in range(_BINS):
            c_k = cacc[k]
            a_k = acc_ref[k]
            a_new = jnp.where(c_k > 0, _MOM * a_k + (np.float32(1) - _MOM) * c_k, a_k)
            w_k = jnp.where(c_k > 0, total / a_new, np.float32(0))
            loss = loss + w_k * sacc[k]
            wsum = wsum + w_k * c_k
        n_elems = np.float32(nblk * rows)
        out_ref[...] = jnp.reshape(loss / wsum * (total / n_elems), (1, 1))


def kernel(logits, labels, acc_sum):
    n, c = logits.shape
    rows = 1024
    nblk = n // rows
    labels3 = labels.reshape(nblk, 1, rows)
    total = np.float32(n)
    body = functools.partial(_ghm_body, nblk=nblk, rows=rows, ncls=c, total=total)
    out = pl.pallas_call(
        body,
        grid=(nblk,),
        in_specs=[
            pl.BlockSpec((rows, c), lambda i: (i, 0)),
            pl.BlockSpec((1, 1, rows), lambda i: (i, 0, 0)),
            pl.BlockSpec(memory_space=pltpu.SMEM),
        ],
        out_specs=pl.BlockSpec((1, 1), lambda i: (0, 0)),
        out_shape=jax.ShapeDtypeStruct((1, 1), jnp.float32),
        scratch_shapes=[
            pltpu.SMEM((_BINS,), jnp.float32),
            pltpu.SMEM((_BINS,), jnp.float32),
        ],
        compiler_params=pltpu.CompilerParams(dimension_semantics=("arbitrary",)),
    )(logits, labels3, acc_sum)
    return out[0, 0]


# speedup vs baseline: 1.2752x; 1.2752x over previous
import functools

import numpy as np
import jax
import jax.numpy as jnp
from jax import lax
from jax.experimental import pallas as pl
from jax.experimental.pallas import tpu as pltpu

_BINS = 10
_MOM = np.float32(0.75)


def _ghm_body(logits_ref, labels_ref, acc_ref, out_ref, g_scr, ce_scr,
              *, nblk, rows, ncls, total):
    i = pl.program_id(0)
    x = logits_ref[...]
    lab = labels_ref[0, 0, :]
    m = jnp.max(x, axis=1)
    e = jnp.exp(x - m[:, None])
    z = jnp.sum(e, axis=1)
    col = lax.broadcasted_iota(jnp.int32, (rows, ncls), 1)
    sel = col == lab[:, None]
    xl = jnp.sum(jnp.where(sel, x, np.float32(0)), axis=1)
    u = xl - m
    ce = jnp.log(z) - u
    g = np.float32(1) - jnp.exp(u) / z
    g_scr[pl.ds(i, 1), :] = g.reshape(1, rows)
    ce_scr[pl.ds(i, 1), :] = ce.reshape(1, rows)

    @pl.when(i == nblk - 1)
    def _finish():
        gg = g_scr[...]
        cc = ce_scr[...]
        binv = jnp.zeros(gg.shape, jnp.int32)
        for j in range(_BINS):
            binv = binv + (gg > np.float32(j) / np.float32(10)).astype(jnp.int32)
        w = jnp.zeros(gg.shape, jnp.float32)
        for k in range(_BINS):
            mk = binv == k
            c_k = jnp.sum(mk.astype(jnp.float32))
            a_k = acc_ref[k]
            a_new = jnp.where(c_k > 0, _MOM * a_k + (np.float32(1) - _MOM) * c_k, a_k)
            w_k = jnp.where(c_k > 0, total / a_new, np.float32(0))
            w = w + jnp.where(mk, w_k, np.float32(0))
        wsum = jnp.sum(w)
        loss = jnp.sum(cc * w)
        n_elems = np.float32(nblk * rows)
        out_ref[...] = jnp.reshape(loss / wsum * (total / n_elems), (1, 1))


def kernel(logits, labels, acc_sum):
    n, c = logits.shape
    rows = 1024
    nblk = n // rows
    labels3 = labels.reshape(nblk, 1, rows)
    total = np.float32(n)
    body = functools.partial(_ghm_body, nblk=nblk, rows=rows, ncls=c, total=total)
    out = pl.pallas_call(
        body,
        grid=(nblk,),
        in_specs=[
            pl.BlockSpec((rows, c), lambda i: (i, 0)),
            pl.BlockSpec((1, 1, rows), lambda i: (i, 0, 0)),
            pl.BlockSpec(memory_space=pltpu.SMEM),
        ],
        out_specs=pl.BlockSpec((1, 1), lambda i: (0, 0)),
        out_shape=jax.ShapeDtypeStruct((1, 1), jnp.float32),
        scratch_shapes=[
            pltpu.VMEM((nblk, rows), jnp.float32),
            pltpu.VMEM((nblk, rows), jnp.float32),
        ],
        compiler_params=pltpu.CompilerParams(dimension_semantics=("arbitrary",)),
    )(logits, labels3, acc_sum)
    return out[0, 0]
